# Initial kernel scaffold; baseline (speedup 1.0000x reference)
import jax
import jax.numpy as jnp
from jax.experimental import pallas as pl


def kernel(x_emb1, x_emb2, edge_e1, edge_e2, W1, b1, W2, b2, gamma, beta, W_feat, b_feat, W_proj, b_proj, x, edge_index, edge_attr, batch):
    raise NotImplementedError("write your pallas kernel here")


# speedup vs baseline: 10.4794x; 10.4794x over previous
import functools

import jax
import jax.numpy as jnp
from jax import lax
from jax.experimental import pallas as pl
from jax.experimental.pallas import tpu as pltpu
from jax.experimental.pallas import tpu_sc as plsc

N = 10000
E = 160000
D = 300
L = 5
G = 128
FEAT = 512
PROJ = 256

HALF = 150
DH = 160
NC, NS = 2, 16
CH = 80
KT = 125
BC = 25
NB = KT // BC
BLK = 1000


_mesh = plsc.VectorSubcoreMesh(core_axis_name="c", subcore_axis_name="s")
_sc_params = pltpu.CompilerParams(use_tc_tiling_on_sc=False,
                                  needs_layout_passes=False)


def _zero_acc(zrows, acc, sid):
    @pl.when(sid < NS - 1)
    def _():
        zbase = sid * 624
        for z in range(4):
            pltpu.sync_copy(zrows.at[pl.ds(0, 128)],
                            acc.at[pl.ds(zbase + 128 * z, 128)])
        pltpu.sync_copy(zrows.at[pl.ds(0, 112)],
                        acc.at[pl.ds(zbase + 512, 112)])

    @pl.when(sid == NS - 1)
    def _():
        for z in range(5):
            pltpu.sync_copy(zrows.at[pl.ds(0, 128)],
                            acc.at[pl.ds(624 * (NS - 1) + 128 * z, 128)])


def _copy_out(acc, out, sid):
    @pl.when(sid < NS - 1)
    def _():
        obase = sid * 624
        pltpu.sync_copy(acc.at[pl.ds(obase, 624)], out.at[pl.ds(obase, 624)])

    @pl.when(sid == NS - 1)
    def _():
        pltpu.sync_copy(acc.at[pl.ds(624 * (NS - 1), 640)],
                        out.at[pl.ds(624 * (NS - 1), 640)])


def _sc_agg_body(hA, hB, gidx, sidx, zrows, aggA, aggB,
                 idxg_v, idxs_v, buf0, buf1, acc, sem0, sem1):
    cid = lax.axis_index("c")
    sid = lax.axis_index("s")
    _zero_acc(zrows, acc, sid)
    plsc.subcore_barrier()
    bufs = (buf0, buf1)
    sems = (sem0, sem1)

    def edge_loop(tbl, out):
        def block(b, carry):
            pltpu.sync_copy(gidx.at[sid, pl.ds(b * BC, BC)], idxg_v)
            pltpu.sync_copy(sidx.at[sid, pl.ds(b * BC, BC)], idxs_v)
            pltpu.async_copy(tbl.at[idxg_v.at[0]], bufs[0], sems[0])
            for t in range(BC):
                if t + 1 < BC:
                    pltpu.async_copy(tbl.at[idxg_v.at[t + 1]],
                                     bufs[(t + 1) % 2], sems[(t + 1) % 2])
                pltpu.make_async_copy(tbl.at[idxg_v.at[t]],
                                      bufs[t % 2], sems[t % 2]).wait()
                pltpu.sync_copy(bufs[t % 2], acc.at[idxs_v.at[t]], add=True)
            return carry

        lax.fori_loop(0, NB, block, 0)
        plsc.subcore_barrier()
        _copy_out(acc, out, sid)

    @pl.when(cid == 0)
    def _():
        edge_loop(hA, aggA)

    @pl.when(cid == 1)
    def _():
        edge_loop(hB, aggB)


def _sc_agg(hA, hB, gidx, sidx, zrows):
    f = pl.kernel(
        _sc_agg_body,
        out_type=(jax.ShapeDtypeStruct((N, DH), jnp.float32),
                  jax.ShapeDtypeStruct((N, DH), jnp.float32)),
        mesh=_mesh,
        scratch_types=[
            pltpu.VMEM((BC, CH), jnp.int32),
            pltpu.VMEM((BC, CH), jnp.int32),
            pltpu.VMEM((CH, DH), jnp.float32),
            pltpu.VMEM((CH, DH), jnp.float32),
            pltpu.VMEM_SHARED((N, DH), jnp.float32),
            pltpu.SemaphoreType.DMA,
            pltpu.SemaphoreType.DMA,
        ],
        compiler_params=_sc_params,
    )
    return f(hA, hB, gidx, sidx, zrows)


def _sc_counts_body(sidx, cidx, z16, counts,
                    idxs_v, idxc_v, buf, acc, sem0):
    cid = lax.axis_index("c")
    sid = lax.axis_index("s")

    @pl.when(cid == 0)
    def _():
        _zero_acc(z16, acc, sid)
        plsc.subcore_barrier()

        ones16 = jnp.ones((16,), jnp.float32)

        def block(b, carry):
            pltpu.sync_copy(sidx.at[sid, pl.ds(b * BC, BC)], idxs_v)
            pltpu.sync_copy(cidx.at[sid, pl.ds(b * BC, BC)], idxc_v)
            for t in range(BC):
                pltpu.sync_copy(z16.at[pl.ds(0, CH)], buf)
                for i in range(CH // 16):
                    c16 = idxc_v[t, pl.ds(i * 16, 16)]
                    e16 = lax.iota(jnp.int32, 16) + i * 16
                    plsc.store_scatter(buf, [e16, c16], ones16)
                pltpu.sync_copy(buf, acc.at[idxs_v.at[t]], add=True)
            return carry

        lax.fori_loop(0, NB, block, 0)
        plsc.subcore_barrier()
        _copy_out(acc, counts, sid)


def _sc_counts(sidx, cidx, z16):
    f = pl.kernel(
        _sc_counts_body,
        out_type=jax.ShapeDtypeStruct((N, 16), jnp.float32),
        mesh=_mesh,
        scratch_types=[
            pltpu.VMEM((BC, CH), jnp.int32),
            pltpu.VMEM((BC, CH), jnp.int32),
            pltpu.VMEM((CH, 16), jnp.float32),
            pltpu.VMEM_SHARED((N, 16), jnp.float32),
            pltpu.SemaphoreType.DMA,
        ],
        compiler_params=_sc_params,
    )
    return f(sidx, cidx, z16)


def _tc_init_kernel(x0_ref, x1_ref, e1A_ref, e1B_ref, e2A_ref, e2B_ref,
                    hA_ref, hB_ref):
    x0 = x0_ref[0, 0, :][:, None]
    x1 = x1_ref[0, 0, :][:, None]
    oh1 = (lax.broadcasted_iota(jnp.int32, (BLK, 128), 1) == x0
           ).astype(jnp.float32)
    oh2 = (lax.broadcasted_iota(jnp.int32, (BLK, 8), 1) == x1
           ).astype(jnp.float32)
    hx = lax.Precision.HIGHEST
    hA_ref[...] = (jnp.dot(oh1, e1A_ref[...], preferred_element_type=jnp.float32, precision=hx)
                   + jnp.dot(oh2, e2A_ref[...], preferred_element_type=jnp.float32, precision=hx))
    hB_ref[...] = (jnp.dot(oh1, e1B_ref[...], preferred_element_type=jnp.float32, precision=hx)
                   + jnp.dot(oh2, e2B_ref[...], preferred_element_type=jnp.float32, precision=hx))


def _tc_init(x0, x1, e1A, e1B, e2A, e2B):
    nb = N // BLK
    full = lambda shp: pl.BlockSpec(shp, lambda i: (0, 0))
    return pl.pallas_call(
        _tc_init_kernel,
        grid=(nb,),
        in_specs=[
            pl.BlockSpec((1, 1, BLK), lambda i: (i, 0, 0)),
            pl.BlockSpec((1, 1, BLK), lambda i: (i, 0, 0)),
            full((128, DH)), full((128, DH)), full((8, DH)), full((8, DH)),
        ],
        out_specs=[
            pl.BlockSpec((BLK, DH), lambda i: (i, 0)),
            pl.BlockSpec((BLK, DH), lambda i: (i, 0)),
        ],
        out_shape=[jax.ShapeDtypeStruct((N, DH), jnp.float32)] * 2,
    )(x0, x1, e1A, e1B, e2A, e2B)


def _tc_mlp_kernel(aggA_ref, aggB_ref, hA_ref, hB_ref, cnt_ref,
                   cmbA_ref, cmbB_ref, W1A_ref, W1B_ref, b1_ref,
                   W2A_ref, W2B_ref, b2A_ref, b2B_ref,
                   h2A_ref, h2B_ref, stA_ref, stB_ref, accA, accB):
    i = pl.program_id(0)
    cnt = cnt_ref[...]
    hx = lax.Precision.HIGHEST
    zA = (aggA_ref[...] + hA_ref[...]
          + jnp.dot(cnt, cmbA_ref[...], preferred_element_type=jnp.float32, precision=hx)
          + cmbA_ref[15:16, :])
    zB = (aggB_ref[...] + hB_ref[...]
          + jnp.dot(cnt, cmbB_ref[...], preferred_element_type=jnp.float32, precision=hx)
          + cmbB_ref[15:16, :])
    U = (jnp.dot(zA, W1A_ref[...], preferred_element_type=jnp.float32)
         + jnp.dot(zB, W1B_ref[...], preferred_element_type=jnp.float32)
         + b1_ref[...])
    U = jnp.maximum(U, 0.0)
    h2A = jnp.dot(U, W2A_ref[...], preferred_element_type=jnp.float32) + b2A_ref[...]
    h2B = jnp.dot(U, W2B_ref[...], preferred_element_type=jnp.float32) + b2B_ref[...]
    h2A_ref[...] = h2A
    h2B_ref[...] = h2B

    @pl.when(i == 0)
    def _():
        accA[...] = jnp.zeros((8, DH), jnp.float32)
        accB[...] = jnp.zeros((8, DH), jnp.float32)

    accA[0:1, :] = accA[0:1, :] + jnp.sum(h2A, axis=0, keepdims=True)
    accA[1:2, :] = accA[1:2, :] + jnp.sum(h2A * h2A, axis=0, keepdims=True)
    accB[0:1, :] = accB[0:1, :] + jnp.sum(h2B, axis=0, keepdims=True)
    accB[1:2, :] = accB[1:2, :] + jnp.sum(h2B * h2B, axis=0, keepdims=True)

    @pl.when(i == pl.num_programs(0) - 1)
    def _():
        stA_ref[...] = accA[...]
        stB_ref[...] = accB[...]


def _tc_mlp(aggA, aggB, hA, hB, cnt, cmbA, cmbB, W1A, W1B, b1,
            W2A, W2B, b2A, b2B):
    nb = N // BLK
    blk = lambda: pl.BlockSpec((BLK, DH), lambda i: (i, 0))
    full = lambda shp: pl.BlockSpec(shp, lambda i: tuple(0 for _ in shp))
    return pl.pallas_call(
        _tc_mlp_kernel,
        grid=(nb,),
        in_specs=[
            blk(), blk(), blk(), blk(),
            pl.BlockSpec((BLK, 16), lambda i: (i, 0)),
            full((16, DH)), full((16, DH)),
            full((DH, 2 * D)), full((DH, 2 * D)), full((1, 2 * D)),
            full((2 * D, DH)), full((2 * D, DH)), full((1, DH)), full((1, DH)),
        ],
        out_specs=[
            pl.BlockSpec((BLK, DH), lambda i: (i, 0)),
            pl.BlockSpec((BLK, DH), lambda i: (i, 0)),
            pl.BlockSpec((8, DH), lambda i: (0, 0)),
            pl.BlockSpec((8, DH), lambda i: (0, 0)),
        ],
        out_shape=[
            jax.ShapeDtypeStruct((N, DH), jnp.float32),
            jax.ShapeDtypeStruct((N, DH), jnp.float32),
            jax.ShapeDtypeStruct((8, DH), jnp.float32),
            jax.ShapeDtypeStruct((8, DH), jnp.float32),
        ],
        scratch_shapes=[pltpu.VMEM((8, DH), jnp.float32)] * 2,
    )(aggA, aggB, hA, hB, cnt, cmbA, cmbB, W1A, W1B, b1, W2A, W2B, b2A, b2B)


def _tc_norm_kernel(h2A_ref, h2B_ref, stA_ref, stB_ref,
                    gA_ref, bA_ref, gB_ref, bB_ref, hA_ref, hB_ref,
                    *, do_relu):
    n = jnp.float32(N)

    def norm(h2_ref, st_ref, g_ref, b_ref, out_ref):
        mu = st_ref[0:1, :] / n
        var = st_ref[1:2, :] / n - mu * mu
        scale = g_ref[...] * lax.rsqrt(var + 1e-5)
        out = h2_ref[...] * scale + (b_ref[...] - mu * scale)
        if do_relu:
            out = jnp.maximum(out, 0.0)
        out_ref[...] = out

    norm(h2A_ref, stA_ref, gA_ref, bA_ref, hA_ref)
    norm(h2B_ref, stB_ref, gB_ref, bB_ref, hB_ref)


def _tc_norm(h2A, h2B, stA, stB, gA, bA, gB, bB, do_relu):
    nb = N // BLK
    blk = lambda: pl.BlockSpec((BLK, DH), lambda i: (i, 0))
    full = lambda shp: pl.BlockSpec(shp, lambda i: tuple(0 for _ in shp))
    return pl.pallas_call(
        functools.partial(_tc_norm_kernel, do_relu=do_relu),
        grid=(nb,),
        in_specs=[
            blk(), blk(),
            full((8, DH)), full((8, DH)),
            full((1, DH)), full((1, DH)), full((1, DH)), full((1, DH)),
        ],
        out_specs=[
            pl.BlockSpec((BLK, DH), lambda i: (i, 0)),
            pl.BlockSpec((BLK, DH), lambda i: (i, 0)),
        ],
        out_shape=[jax.ShapeDtypeStruct((N, DH), jnp.float32)] * 2,
    )(h2A, h2B, stA, stB, gA, bA, gB, bB)


def _tc_pool_kernel(hA_ref, hB_ref, bt_ref, WfA_ref, WfB_ref, bf_ref,
                    Wp_ref, bp_ref, out_ref, pA, pB, cnt):
    i = pl.program_id(0)

    @pl.when(i == 0)
    def _():
        pA[...] = jnp.zeros((G, DH), jnp.float32)
        pB[...] = jnp.zeros((G, DH), jnp.float32)
        cnt[...] = jnp.zeros((G, 128), jnp.float32)

    b = bt_ref[0]
    m = (lax.broadcasted_iota(jnp.int32, (G, BLK), 0) == b).astype(jnp.float32)
    hx = lax.Precision.HIGHEST
    pA[...] = pA[...] + jnp.dot(m, hA_ref[...], preferred_element_type=jnp.float32, precision=hx)
    pB[...] = pB[...] + jnp.dot(m, hB_ref[...], preferred_element_type=jnp.float32, precision=hx)
    cnt[...] = cnt[...] + jnp.sum(m, axis=1, keepdims=True)

    @pl.when(i == pl.num_programs(0) - 1)
    def _():
        c = jnp.maximum(cnt[:, 0:1], 1.0)
        pooledA = pA[...] / c
        pooledB = pB[...] / c
        feat = (jnp.dot(pooledA, WfA_ref[...], preferred_element_type=jnp.float32)
                + jnp.dot(pooledB, WfB_ref[...], preferred_element_type=jnp.float32)
                + bf_ref[...])
        out_ref[...] = (jnp.dot(feat, Wp_ref[...], preferred_element_type=jnp.float32)
                        + bp_ref[...])


def _tc_pool(hA, hB, bt, WfA, WfB, bf, Wp, bp):
    nb = N // BLK
    blk = lambda: pl.BlockSpec((BLK, DH), lambda i: (i, 0))
    full = lambda shp: pl.BlockSpec(shp, lambda i: tuple(0 for _ in shp))
    return pl.pallas_call(
        _tc_pool_kernel,
        grid=(nb,),
        in_specs=[
            blk(), blk(),
            pl.BlockSpec((1, 1, BLK), lambda i: (i, 0, 0)),
            full((DH, FEAT)), full((DH, FEAT)), full((1, FEAT)),
            full((FEAT, PROJ)), full((1, PROJ)),
        ],
        out_specs=pl.BlockSpec((G, PROJ), lambda i: (0, 0)),
        out_shape=jax.ShapeDtypeStruct((G, PROJ), jnp.float32),
        scratch_shapes=[
            pltpu.VMEM((G, DH), jnp.float32),
            pltpu.VMEM((G, DH), jnp.float32),
            pltpu.VMEM((G, 128), jnp.float32),
        ],
    )(hA, hB, bt, WfA, WfB, bf, Wp, bp)


def _pad_cols(w):
    pads = [(0, 0)] * (w.ndim - 1)
    a = jnp.pad(w[..., :HALF], pads + [(0, DH - HALF)])
    b = jnp.pad(w[..., HALF:], pads + [(0, DH - HALF)])
    return a, b


def _pad_rows(w):
    pads = [(0, 0)] * (w.ndim - 1)
    a = jnp.pad(w[:HALF], [(0, DH - HALF)] + pads)
    b = jnp.pad(w[HALF:], [(0, DH - HALF)] + pads)
    return a, b


def kernel(x_emb1, x_emb2, edge_e1, edge_e2, W1, b1, W2, b2, gamma, beta,
           W_feat, b_feat, W_proj, b_proj, x, edge_index, edge_attr, batch):
    combo_id = edge_attr[:, 0] * 3 + edge_attr[:, 1]
    sidx = edge_index[0].reshape(NS, KT, CH)
    gidx = edge_index[1].reshape(NS, KT, CH)
    cidx = combo_id.reshape(NS, KT, CH)
    zrows = jnp.zeros((128, DH), jnp.float32)
    z16 = jnp.zeros((128, 16), jnp.float32)
    x0 = x[:, 0].reshape(N // BLK, 1, BLK)
    x1 = x[:, 1].reshape(N // BLK, 1, BLK)
    bt = batch.reshape(N // BLK, 1, BLK)

    e1A, e1B = _pad_cols(jnp.pad(x_emb1, ((0, 128 - 119), (0, 0))))
    e2A, e2B = _pad_cols(jnp.pad(x_emb2, ((0, 8 - 3), (0, 0))))
    ia0 = jnp.repeat(jnp.arange(5), 3)
    ia1 = jnp.tile(jnp.arange(3), 5)
    combo = edge_e1[:, ia0, :] + edge_e2[:, ia1, :]
    selfv = (edge_e1[:, 4, :] + edge_e2[:, 0, :])[:, None, :]
    combo = jnp.concatenate([combo, selfv], axis=1)
    cmbA, cmbB = _pad_cols(combo)
    W1A, W1B = _pad_rows(W1.transpose(1, 0, 2).reshape(D, L * 2 * D))
    W1A = W1A.reshape(DH, L, 2 * D).transpose(1, 0, 2)
    W1B = W1B.reshape(DH, L, 2 * D).transpose(1, 0, 2)
    W2A, W2B = _pad_cols(W2)
    b2A, b2B = _pad_cols(b2)
    gA, gB = _pad_cols(gamma)
    bA, bB = _pad_cols(beta)
    WfA, WfB = _pad_rows(W_feat)

    cnt = _sc_counts(sidx, cidx, z16)
    hA, hB = _tc_init(x0, x1, e1A, e1B, e2A, e2B)
    for l in range(L):
        aggA, aggB = _sc_agg(hA, hB, gidx, sidx, zrows)
        h2A, h2B, stA, stB = _tc_mlp(
            aggA, aggB, hA, hB, cnt, cmbA[l], cmbB[l],
            W1A[l], W1B[l], b1[l].reshape(1, 2 * D),
            W2A[l], W2B[l], b2A[l].reshape(1, DH), b2B[l].reshape(1, DH))
        hA, hB = _tc_norm(h2A, h2B, stA, stB,
                          gA[l].reshape(1, DH), bA[l].reshape(1, DH),
                          gB[l].reshape(1, DH), bB[l].reshape(1, DH),
                          do_relu=(l != L - 1))
    return _tc_pool(hA, hB, bt, WfA, WfB, b_feat.reshape(1, FEAT),
                    W_proj, b_proj.reshape(1, PROJ))
